# baseline (speedup 1.0000x reference)
import functools

import jax
import jax.numpy as jnp
from jax import lax
from jax.experimental import pallas as pl
from jax.experimental.pallas import tpu as pltpu
from jax.experimental.pallas import tpu_sc as plsc

_CHUNK = 128
_NBUF = 4


def _gather3(B, D, NC, NS):
    NW = NC * NS
    b_per_w = B // NW
    n_chunks = b_per_w // _CHUNK
    mesh = plsc.VectorSubcoreMesh(core_axis_name="c", subcore_axis_name="s")

    scratch = (
        [pltpu.VMEM((b_per_w,), jnp.int32) for _ in range(3)]
        + [pltpu.VMEM((_CHUNK, D), jnp.float32) for _ in range(_NBUF)]
        + [pltpu.SemaphoreType.DMA for _ in range(2 * _NBUF)]
    )

    @functools.partial(
        pl.kernel,
        mesh=mesh,
        out_type=(
            jax.ShapeDtypeStruct((B, D), jnp.float32),
            jax.ShapeDtypeStruct((B, D), jnp.float32),
            jax.ShapeDtypeStruct((B, D), jnp.float32),
        ),
        scratch_types=scratch,
    )
    def k(emb_hbm, kgg_hbm, rel_hbm, scg_ids_hbm, kgg_ids_hbm, rel_ids_hbm,
          out_scg, out_kgg, out_rel, *sc):
        idxs = sc[0:3]
        bufs = sc[3:3 + _NBUF]
        gsems = sc[3 + _NBUF:3 + 2 * _NBUF]
        ssems = sc[3 + 2 * _NBUF:3 + 3 * _NBUF]

        wid = lax.axis_index("s") * NC + lax.axis_index("c")
        base = wid * b_per_w

        for ids_hbm, idx_v in zip(
                (scg_ids_hbm, kgg_ids_hbm, rel_ids_hbm), idxs):
            pltpu.sync_copy(ids_hbm.at[pl.ds(base, b_per_w)], idx_v)

        work = []
        for t, (table_hbm, out_hbm, idx_v) in enumerate((
                (emb_hbm, out_scg, idxs[0]),
                (kgg_hbm, out_kgg, idxs[1]),
                (rel_hbm, out_rel, idxs[2]),
        )):
            for c in range(n_chunks):
                work.append((table_hbm, out_hbm, idx_v, c))

        n = len(work)
        gathers = [None] * n
        scatters = [None] * n

        def issue_gather(i):
            table_hbm, _, idx_v, c = work[i]
            b = i % _NBUF
            if i - _NBUF >= 0:
                scatters[i - _NBUF].wait()
            gathers[i] = pltpu.async_copy(
                table_hbm.at[idx_v.at[pl.ds(c * _CHUNK, _CHUNK)]],
                bufs[b], gsems[b])

        issue_gather(0)
        for i in range(n):
            if i + 1 < n:
                issue_gather(i + 1)
            _, out_hbm, _, c = work[i]
            b = i % _NBUF
            gathers[i].wait()
            scatters[i] = pltpu.async_copy(
                bufs[b], out_hbm.at[pl.ds(base + c * _CHUNK, _CHUNK)],
                ssems[b])
        for i in range(max(0, n - _NBUF), n):
            scatters[i].wait()

    return k


def kernel(embedding, kgg_table, rel_table, scg_ids, relation_ids, kgg_ids):
    B = scg_ids.shape[0]
    D = embedding.shape[1]
    info = plsc.get_sparse_core_info()
    NC, NS = info.num_cores, info.num_subcores
    k = _gather3(B, D, NC, NS)
    scg, kgg, rel = k(
        embedding,
        kgg_table,
        rel_table,
        scg_ids.astype(jnp.int32),
        kgg_ids.astype(jnp.int32),
        relation_ids.astype(jnp.int32),
    )
    return (scg, kgg, rel)


# speedup vs baseline: 1.0203x; 1.0203x over previous
import functools

import jax
import jax.numpy as jnp
from jax import lax
from jax.experimental import pallas as pl
from jax.experimental.pallas import tpu as pltpu
from jax.experimental.pallas import tpu_sc as plsc

_CHUNK = 256
_NBUF = 3


def _gather3(B, D, NC, NS):
    NW = NC * NS
    b_per_w = B // NW
    n_chunks = b_per_w // _CHUNK
    mesh = plsc.VectorSubcoreMesh(core_axis_name="c", subcore_axis_name="s")

    scratch = (
        [pltpu.VMEM((b_per_w,), jnp.int32) for _ in range(3)]
        + [pltpu.VMEM((_CHUNK, D), jnp.float32) for _ in range(_NBUF)]
        + [pltpu.SemaphoreType.DMA for _ in range(2 * _NBUF)]
    )

    @functools.partial(
        pl.kernel,
        mesh=mesh,
        out_type=(
            jax.ShapeDtypeStruct((B, D), jnp.float32),
            jax.ShapeDtypeStruct((B, D), jnp.float32),
            jax.ShapeDtypeStruct((B, D), jnp.float32),
        ),
        scratch_types=scratch,
    )
    def k(emb_hbm, kgg_hbm, rel_hbm, scg_ids_hbm, kgg_ids_hbm, rel_ids_hbm,
          out_scg, out_kgg, out_rel, *sc):
        idxs = sc[0:3]
        bufs = sc[3:3 + _NBUF]
        gsems = sc[3 + _NBUF:3 + 2 * _NBUF]
        ssems = sc[3 + 2 * _NBUF:3 + 3 * _NBUF]

        wid = lax.axis_index("s") * NC + lax.axis_index("c")
        base = wid * b_per_w

        for ids_hbm, idx_v in zip(
                (scg_ids_hbm, kgg_ids_hbm, rel_ids_hbm), idxs):
            pltpu.sync_copy(ids_hbm.at[pl.ds(base, b_per_w)], idx_v)

        work = []
        for t, (table_hbm, out_hbm, idx_v) in enumerate((
                (emb_hbm, out_scg, idxs[0]),
                (kgg_hbm, out_kgg, idxs[1]),
                (rel_hbm, out_rel, idxs[2]),
        )):
            for c in range(n_chunks):
                work.append((table_hbm, out_hbm, idx_v, c))

        n = len(work)
        gathers = [None] * n
        scatters = [None] * n

        def issue_gather(i):
            table_hbm, _, idx_v, c = work[i]
            b = i % _NBUF
            if i - _NBUF >= 0:
                scatters[i - _NBUF].wait()
            gathers[i] = pltpu.async_copy(
                table_hbm.at[idx_v.at[pl.ds(c * _CHUNK, _CHUNK)]],
                bufs[b], gsems[b])

        issue_gather(0)
        for i in range(n):
            if i + 1 < n:
                issue_gather(i + 1)
            _, out_hbm, _, c = work[i]
            b = i % _NBUF
            gathers[i].wait()
            scatters[i] = pltpu.async_copy(
                bufs[b], out_hbm.at[pl.ds(base + c * _CHUNK, _CHUNK)],
                ssems[b])
        for i in range(max(0, n - _NBUF), n):
            scatters[i].wait()

    return k


def kernel(embedding, kgg_table, rel_table, scg_ids, relation_ids, kgg_ids):
    B = scg_ids.shape[0]
    D = embedding.shape[1]
    info = plsc.get_sparse_core_info()
    NC, NS = info.num_cores, info.num_subcores
    k = _gather3(B, D, NC, NS)
    scg, kgg, rel = k(
        embedding,
        kgg_table,
        rel_table,
        scg_ids.astype(jnp.int32),
        kgg_ids.astype(jnp.int32),
        relation_ids.astype(jnp.int32),
    )
    return (scg, kgg, rel)
